# baseline (speedup 1.0000x reference)
import functools

import jax
import jax.numpy as jnp
from jax import lax
from jax.experimental import pallas as pl
from jax.experimental.pallas import tpu as pltpu
from jax.experimental.pallas import tpu_sc as plsc

TOKENS = 200000
D = 128
CHUNK = 128
NUM_CHUNKS = TOKENS // CHUNK
TAIL = TOKENS - NUM_CHUNKS * CHUNK
TAIL_BASE = NUM_CHUNKS * CHUNK
NC, NS = 2, 16
NW = NC * NS


def _body(x_hbm, table_hbm, pos_hbm, out_hbm,
          xbuf, rowsbuf, idxbuf, xt, rt, it,
          sem_x, sem_i, sem_g, sem_o):
    wid = lax.axis_index("s") * NC + lax.axis_index("c")
    nk = (NUM_CHUNKS - wid + NW - 1) // NW

    def chunk_body(k, carry):
        c = k * NW + wid
        base = c * CHUNK
        cp_x = pltpu.async_copy(x_hbm.at[pl.ds(base, CHUNK)], xbuf, sem_x)
        cp_i = pltpu.async_copy(pos_hbm.at[pl.ds(base, CHUNK)], idxbuf, sem_i)
        cp_i.wait()
        pltpu.async_copy(table_hbm.at[idxbuf], rowsbuf, sem_g).wait()
        cp_x.wait()

        def add_body(t, c2):
            for j in range(D // 16):
                sl = pl.ds(j * 16, 16)
                xbuf[t, sl] = xbuf[t, sl] + rowsbuf[t, sl]
            return c2
        lax.fori_loop(0, CHUNK, add_body, 0, unroll=2)

        pltpu.async_copy(xbuf, out_hbm.at[pl.ds(base, CHUNK)], sem_o).wait()
        return carry

    lax.fori_loop(0, nk, chunk_body, 0)

    @pl.when(wid == NW - 1)
    def _tail():
        cp_x = pltpu.async_copy(x_hbm.at[pl.ds(TAIL_BASE, TAIL)], xt, sem_x)
        cp_i = pltpu.async_copy(pos_hbm.at[pl.ds(TAIL_BASE, TAIL)], it, sem_i)
        cp_i.wait()
        pltpu.async_copy(table_hbm.at[it], rt, sem_g).wait()
        cp_x.wait()

        def add_body(t, c2):
            for j in range(D // 16):
                sl = pl.ds(j * 16, 16)
                xt[t, sl] = xt[t, sl] + rt[t, sl]
            return c2
        lax.fori_loop(0, TAIL, add_body, 0, unroll=2)

        pltpu.async_copy(xt, out_hbm.at[pl.ds(TAIL_BASE, TAIL)], sem_o).wait()


@jax.jit
def kernel(x, emb_table, pos_matrix):
    x2 = x.reshape(TOKENS, D)
    pos1 = pos_matrix.reshape(TOKENS)
    mesh = plsc.VectorSubcoreMesh(core_axis_name="c", subcore_axis_name="s")
    run = functools.partial(
        pl.kernel,
        mesh=mesh,
        out_type=jax.ShapeDtypeStruct((TOKENS, D), jnp.float32),
        scratch_types=[
            pltpu.VMEM((CHUNK, D), jnp.float32),
            pltpu.VMEM((CHUNK, D), jnp.float32),
            pltpu.VMEM((CHUNK,), jnp.int32),
            pltpu.VMEM((TAIL, D), jnp.float32),
            pltpu.VMEM((TAIL, D), jnp.float32),
            pltpu.VMEM((TAIL,), jnp.int32),
            pltpu.SemaphoreType.DMA,
            pltpu.SemaphoreType.DMA,
            pltpu.SemaphoreType.DMA,
            pltpu.SemaphoreType.DMA,
        ],
    )(_body)
    out2 = run(x2, emb_table, pos1)
    return out2.reshape(x.shape)


# speedup vs baseline: 2.9788x; 2.9788x over previous
import functools

import jax
import jax.numpy as jnp
from jax import lax
from jax.experimental import pallas as pl
from jax.experimental.pallas import tpu as pltpu
from jax.experimental.pallas import tpu_sc as plsc

TOKENS = 200000
D = 128
L = 16
CHUNK = 128
NUM_CHUNKS = TOKENS // CHUNK
TAIL = TOKENS - NUM_CHUNKS * CHUNK
TAIL_BASE = NUM_CHUNKS * CHUNK
NC, NS = 2, 16
NW = NC * NS
KFULL = 48
EXTRA = NUM_CHUNKS - KFULL * NW
TABLE_ROWS = 50


def _body(x_hbm, table_hbm, pos_hbm, out_hbm,
          tbuf, xa, xb, ia, ib, xt, it,
          stb, sxa, sxb, sia, sib, soa, sob, sxt, sit, sot):
    wid = lax.axis_index("s") * NC + lax.axis_index("c")

    pltpu.async_copy(table_hbm, tbuf, stb).wait()
    cols = [lax.iota(jnp.int32, L) + j * L for j in range(D // L)]

    def issue_in(k, xref, iref, sx, si):
        b = (k * NW + wid) * CHUNK
        pltpu.async_copy(x_hbm.at[pl.ds(b, CHUNK)], xref, sx)
        pltpu.async_copy(pos_hbm.at[pl.ds(b, CHUNK)], iref, si)

    def wait_in(xref, iref, sx, si):
        pltpu.make_async_copy(x_hbm.at[pl.ds(0, CHUNK)], xref, sx).wait()
        pltpu.make_async_copy(pos_hbm.at[pl.ds(0, CHUNK)], iref, si).wait()

    def issue_out(k, xref, so):
        b = (k * NW + wid) * CHUNK
        pltpu.async_copy(xref, out_hbm.at[pl.ds(b, CHUNK)], so)

    def wait_out(xref, so):
        pltpu.make_async_copy(xref, out_hbm.at[pl.ds(0, CHUNK)], so).wait()

    def add_chunk(xref, iref, ntok):
        def _tok(t, carry):
            posv = plsc.load_gather(iref, [jnp.full((L,), t, jnp.int32)])
            base = posv * D
            for j in range(D // L):
                rows = plsc.load_gather(tbuf, [base + cols[j]])
                plsc.addupdate(xref.at[t, pl.ds(j * L, L)], rows)
            return carry
        lax.fori_loop(0, ntok, _tok, 0)

    issue_in(0, xa, ia, sxa, sia)
    issue_in(1, xb, ib, sxb, sib)

    def pair_body(k2, carry):
        ka = 2 * k2
        kb = ka + 1
        wait_in(xa, ia, sxa, sia)
        add_chunk(xa, ia, CHUNK)
        issue_out(ka, xa, soa)
        wait_in(xb, ib, sxb, sib)
        add_chunk(xb, ib, CHUNK)
        issue_out(kb, xb, sob)

        @pl.when((ka + 2) * NW + wid < NUM_CHUNKS)
        def _():
            wait_out(xa, soa)
            issue_in(ka + 2, xa, ia, sxa, sia)

        @pl.when((kb + 2) * NW + wid < NUM_CHUNKS)
        def _():
            wait_out(xb, sob)
            issue_in(kb + 2, xb, ib, sxb, sib)
        return carry

    lax.fori_loop(0, KFULL // 2, pair_body, 0)

    @pl.when(wid < EXTRA)
    def _extra():
        wait_in(xa, ia, sxa, sia)
        add_chunk(xa, ia, CHUNK)
        issue_out(KFULL, xa, soa)
        wait_out(xa, soa)

    @pl.when(wid >= EXTRA)
    def _drain_a():
        wait_out(xa, soa)

    wait_out(xb, sob)

    @pl.when(wid == NW - 1)
    def _tail():
        cx = pltpu.async_copy(x_hbm.at[pl.ds(TAIL_BASE, TAIL)], xt, sxt)
        ci = pltpu.async_copy(pos_hbm.at[pl.ds(TAIL_BASE, TAIL)], it, sit)
        ci.wait()
        cx.wait()

        add_chunk(xt, it, TAIL)

        pltpu.async_copy(xt, out_hbm.at[pl.ds(TAIL_BASE, TAIL)], sot).wait()


@jax.jit
def kernel(x, emb_table, pos_matrix):
    x2 = x.reshape(TOKENS, D)
    pos1 = pos_matrix.reshape(TOKENS)
    table1 = emb_table.reshape(TABLE_ROWS * D)
    mesh = plsc.VectorSubcoreMesh(core_axis_name="c", subcore_axis_name="s")
    run = functools.partial(
        pl.kernel,
        mesh=mesh,
        out_type=jax.ShapeDtypeStruct((TOKENS, D), jnp.float32),
        compiler_params=pltpu.CompilerParams(needs_layout_passes=False),
        scratch_types=[
            pltpu.VMEM((TABLE_ROWS * D,), jnp.float32),
            pltpu.VMEM((CHUNK, D), jnp.float32),
            pltpu.VMEM((CHUNK, D), jnp.float32),
            pltpu.VMEM((CHUNK,), jnp.int32),
            pltpu.VMEM((CHUNK,), jnp.int32),
            pltpu.VMEM((TAIL, D), jnp.float32),
            pltpu.VMEM((TAIL,), jnp.int32),
        ] + [pltpu.SemaphoreType.DMA] * 10,
    )(_body)
    out2 = run(x2, table1, pos1)
    return out2.reshape(x.shape)


# speedup vs baseline: 3.1060x; 1.0427x over previous
import functools

import jax
import jax.numpy as jnp
from jax import lax
from jax.experimental import pallas as pl
from jax.experimental.pallas import tpu as pltpu
from jax.experimental.pallas import tpu_sc as plsc

TOKENS = 200000
D = 128
L = 16
CHUNK = 128
NUM_CHUNKS = TOKENS // CHUNK
TAIL = TOKENS - NUM_CHUNKS * CHUNK
TAIL_BASE = NUM_CHUNKS * CHUNK
NC, NS = 2, 16
NW = NC * NS
KFULL = 48
EXTRA = NUM_CHUNKS - KFULL * NW
TABLE_ROWS = 50


def _body(x_hbm, table_hbm, pos_hbm, out_hbm,
          tbuf, xa, xb, ia, ib, xt, it,
          stb, sxa, sxb, sia, sib, soa, sob, sxt, sit, sot):
    wid = lax.axis_index("s") * NC + lax.axis_index("c")

    pltpu.async_copy(table_hbm, tbuf, stb).wait()
    cols = [lax.iota(jnp.int32, L) + j * L for j in range(D // L)]

    def issue_in(k, xref, iref, sx, si):
        b = (k * NW + wid) * CHUNK
        pltpu.async_copy(x_hbm.at[pl.ds(b, CHUNK)], xref, sx)
        pltpu.async_copy(pos_hbm.at[pl.ds(b, CHUNK)], iref, si)

    def wait_in(xref, iref, sx, si):
        pltpu.make_async_copy(x_hbm.at[pl.ds(0, CHUNK)], xref, sx).wait()
        pltpu.make_async_copy(pos_hbm.at[pl.ds(0, CHUNK)], iref, si).wait()

    def issue_out(k, xref, so):
        b = (k * NW + wid) * CHUNK
        pltpu.async_copy(xref, out_hbm.at[pl.ds(b, CHUNK)], so)

    def wait_out(xref, so):
        pltpu.make_async_copy(xref, out_hbm.at[pl.ds(0, CHUNK)], so).wait()

    def add_chunk(xref, iref, ntok):
        def _tok(t, carry):
            posv = plsc.load_gather(iref, [jnp.full((L,), t, jnp.int32)])
            base = posv * D
            for j in range(D // L):
                rows = plsc.load_gather(tbuf, [base + cols[j]])
                plsc.addupdate(xref.at[t, pl.ds(j * L, L)], rows)
            return carry
        lax.fori_loop(0, ntok, _tok, 0, unroll=8)

    issue_in(0, xa, ia, sxa, sia)
    issue_in(1, xb, ib, sxb, sib)

    def pair_body(k2, carry):
        ka = 2 * k2
        kb = ka + 1
        wait_in(xa, ia, sxa, sia)
        add_chunk(xa, ia, CHUNK)
        issue_out(ka, xa, soa)
        wait_in(xb, ib, sxb, sib)
        add_chunk(xb, ib, CHUNK)
        issue_out(kb, xb, sob)

        @pl.when((ka + 2) * NW + wid < NUM_CHUNKS)
        def _():
            wait_out(xa, soa)
            issue_in(ka + 2, xa, ia, sxa, sia)

        @pl.when((kb + 2) * NW + wid < NUM_CHUNKS)
        def _():
            wait_out(xb, sob)
            issue_in(kb + 2, xb, ib, sxb, sib)
        return carry

    lax.fori_loop(0, KFULL // 2, pair_body, 0)

    @pl.when(wid < EXTRA)
    def _extra():
        wait_in(xa, ia, sxa, sia)
        add_chunk(xa, ia, CHUNK)
        issue_out(KFULL, xa, soa)
        wait_out(xa, soa)

    @pl.when(wid >= EXTRA)
    def _drain_a():
        wait_out(xa, soa)

    wait_out(xb, sob)

    @pl.when(wid == NW - 1)
    def _tail():
        cx = pltpu.async_copy(x_hbm.at[pl.ds(TAIL_BASE, TAIL)], xt, sxt)
        ci = pltpu.async_copy(pos_hbm.at[pl.ds(TAIL_BASE, TAIL)], it, sit)
        ci.wait()
        cx.wait()

        add_chunk(xt, it, TAIL)

        pltpu.async_copy(xt, out_hbm.at[pl.ds(TAIL_BASE, TAIL)], sot).wait()


@jax.jit
def kernel(x, emb_table, pos_matrix):
    x2 = x.reshape(TOKENS, D)
    pos1 = pos_matrix.reshape(TOKENS)
    table1 = emb_table.reshape(TABLE_ROWS * D)
    mesh = plsc.VectorSubcoreMesh(core_axis_name="c", subcore_axis_name="s")
    run = functools.partial(
        pl.kernel,
        mesh=mesh,
        out_type=jax.ShapeDtypeStruct((TOKENS, D), jnp.float32),
        compiler_params=pltpu.CompilerParams(needs_layout_passes=False),
        scratch_types=[
            pltpu.VMEM((TABLE_ROWS * D,), jnp.float32),
            pltpu.VMEM((CHUNK, D), jnp.float32),
            pltpu.VMEM((CHUNK, D), jnp.float32),
            pltpu.VMEM((CHUNK,), jnp.int32),
            pltpu.VMEM((CHUNK,), jnp.int32),
            pltpu.VMEM((TAIL, D), jnp.float32),
            pltpu.VMEM((TAIL,), jnp.int32),
        ] + [pltpu.SemaphoreType.DMA] * 10,
    )(_body)
    out2 = run(x2, table1, pos1)
    return out2.reshape(x.shape)


# speedup vs baseline: 4.1768x; 1.3447x over previous
import functools

import jax
import jax.numpy as jnp
from jax import lax
from jax.experimental import pallas as pl
from jax.experimental.pallas import tpu as pltpu
from jax.experimental.pallas import tpu_sc as plsc

TOKENS = 200000
D = 128
L = 16
CHUNK = 128
NUM_CHUNKS = TOKENS // CHUNK
TAIL = TOKENS - NUM_CHUNKS * CHUNK
TAIL_BASE = NUM_CHUNKS * CHUNK
NC, NS = 2, 16
NW = NC * NS
KFULL = 48
EXTRA = NUM_CHUNKS - KFULL * NW
TABLE_ROWS = 50


def _body(x_hbm, table_hbm, pos_hbm, out_hbm,
          tbuf, xa, xb, ia, ib, xt, it,
          stb, sxa, sxb, sia, sib, soa, sob, sxt, sit, sot):
    wid = lax.axis_index("s") * NC + lax.axis_index("c")

    pltpu.async_copy(table_hbm, tbuf, stb).wait()
    cols = [lax.iota(jnp.int32, L) + j * L for j in range(D // L)]

    def issue_in(k, xref, iref, sx, si):
        b = (k * NW + wid) * CHUNK
        pltpu.async_copy(x_hbm.at[pl.ds(b, CHUNK)], xref, sx)
        pltpu.async_copy(pos_hbm.at[pl.ds(b, CHUNK)], iref, si)

    def wait_in(xref, iref, sx, si):
        pltpu.make_async_copy(x_hbm.at[pl.ds(0, CHUNK)], xref, sx).wait()
        pltpu.make_async_copy(pos_hbm.at[pl.ds(0, CHUNK)], iref, si).wait()

    def issue_out(k, xref, so):
        b = (k * NW + wid) * CHUNK
        pltpu.async_copy(xref, out_hbm.at[pl.ds(b, CHUNK)], so)

    def wait_out(xref, so):
        pltpu.make_async_copy(xref, out_hbm.at[pl.ds(0, CHUNK)], so).wait()

    def add_chunk(xref, iref, ntok):
        @plsc.parallel_loop(0, ntok, unroll=8)
        def _tok(t):
            posv = plsc.load_gather(iref, [jnp.full((L,), t, jnp.int32)])
            base = posv * D
            for j in range(D // L):
                rows = plsc.load_gather(tbuf, [base + cols[j]])
                plsc.addupdate(xref.at[t, pl.ds(j * L, L)], rows)

    issue_in(0, xa, ia, sxa, sia)
    issue_in(1, xb, ib, sxb, sib)

    def pair_body(k2, carry):
        ka = 2 * k2
        kb = ka + 1
        wait_in(xa, ia, sxa, sia)
        add_chunk(xa, ia, CHUNK)
        issue_out(ka, xa, soa)
        wait_in(xb, ib, sxb, sib)
        add_chunk(xb, ib, CHUNK)
        issue_out(kb, xb, sob)

        @pl.when((ka + 2) * NW + wid < NUM_CHUNKS)
        def _():
            wait_out(xa, soa)
            issue_in(ka + 2, xa, ia, sxa, sia)

        @pl.when((kb + 2) * NW + wid < NUM_CHUNKS)
        def _():
            wait_out(xb, sob)
            issue_in(kb + 2, xb, ib, sxb, sib)
        return carry

    lax.fori_loop(0, KFULL // 2, pair_body, 0)

    @pl.when(wid < EXTRA)
    def _extra():
        wait_in(xa, ia, sxa, sia)
        add_chunk(xa, ia, CHUNK)
        issue_out(KFULL, xa, soa)
        wait_out(xa, soa)

    @pl.when(wid >= EXTRA)
    def _drain_a():
        wait_out(xa, soa)

    wait_out(xb, sob)

    @pl.when(wid == NW - 1)
    def _tail():
        cx = pltpu.async_copy(x_hbm.at[pl.ds(TAIL_BASE, TAIL)], xt, sxt)
        ci = pltpu.async_copy(pos_hbm.at[pl.ds(TAIL_BASE, TAIL)], it, sit)
        ci.wait()
        cx.wait()

        add_chunk(xt, it, TAIL)

        pltpu.async_copy(xt, out_hbm.at[pl.ds(TAIL_BASE, TAIL)], sot).wait()


@jax.jit
def kernel(x, emb_table, pos_matrix):
    x2 = x.reshape(TOKENS, D)
    pos1 = pos_matrix.reshape(TOKENS)
    table1 = emb_table.reshape(TABLE_ROWS * D)
    mesh = plsc.VectorSubcoreMesh(core_axis_name="c", subcore_axis_name="s")
    run = functools.partial(
        pl.kernel,
        mesh=mesh,
        out_type=jax.ShapeDtypeStruct((TOKENS, D), jnp.float32),
        compiler_params=pltpu.CompilerParams(needs_layout_passes=False),
        scratch_types=[
            pltpu.VMEM((TABLE_ROWS * D,), jnp.float32),
            pltpu.VMEM((CHUNK, D), jnp.float32),
            pltpu.VMEM((CHUNK, D), jnp.float32),
            pltpu.VMEM((CHUNK,), jnp.int32),
            pltpu.VMEM((CHUNK,), jnp.int32),
            pltpu.VMEM((TAIL, D), jnp.float32),
            pltpu.VMEM((TAIL,), jnp.int32),
        ] + [pltpu.SemaphoreType.DMA] * 10,
    )(_body)
    out2 = run(x2, table1, pos1)
    return out2.reshape(x.shape)


# speedup vs baseline: 6.8832x; 1.6480x over previous
import functools

import jax
import jax.numpy as jnp
from jax import lax
from jax.experimental import pallas as pl
from jax.experimental.pallas import tpu as pltpu
from jax.experimental.pallas import tpu_sc as plsc

ROWS = 400
COLS = 500
D = 128
L = 16
H1 = 248
H2 = COLS - H1
NC, NS = 2, 16
NW = NC * NS
KFULL = ROWS // NW
EXTRA = ROWS - KFULL * NW
TABLE_ROWS = 50


def _body(x3, table1, pos2, out3,
          tbuf, xa, xb, p0, p1,
          stb, sxa, sxb, sp0, sp1, soa, sob):
    wid = lax.axis_index("s") * NC + lax.axis_index("c")

    pltpu.async_copy(table1, tbuf, stb).wait()
    cols = [lax.iota(jnp.int32, L) + j * L for j in range(D // L)]

    def rowid(k):
        return k * NW + wid

    def issue_xa(k):
        pltpu.async_copy(x3.at[rowid(k), pl.ds(0, H1)], xa, sxa)

    def issue_xb(k):
        pltpu.async_copy(x3.at[rowid(k), pl.ds(H1, H2)], xb, sxb)

    def wait_xa():
        pltpu.make_async_copy(x3.at[0, pl.ds(0, H1)], xa, sxa).wait()

    def wait_xb():
        pltpu.make_async_copy(x3.at[0, pl.ds(H1, H2)], xb, sxb).wait()

    def issue_p(k, pref, sp):
        pltpu.async_copy(pos2.at[rowid(k)], pref, sp)

    def wait_p(pref, sp):
        pltpu.make_async_copy(pos2.at[0], pref, sp).wait()

    def issue_oa(k):
        pltpu.async_copy(xa, out3.at[rowid(k), pl.ds(0, H1)], soa)

    def issue_ob(k):
        pltpu.async_copy(xb, out3.at[rowid(k), pl.ds(H1, H2)], sob)

    def wait_oa():
        pltpu.make_async_copy(xa, out3.at[0, pl.ds(0, H1)], soa).wait()

    def wait_ob():
        pltpu.make_async_copy(xb, out3.at[0, pl.ds(H1, H2)], sob).wait()

    def valid(k):
        return (k < KFULL) | ((k == KFULL) & (wid < EXTRA))

    def add_half(xref, pref, ntok, off):
        @plsc.parallel_loop(0, ntok, unroll=8)
        def _tok(t):
            posv = plsc.load_gather(pref, [jnp.full((L,), t + off, jnp.int32)])
            base = posv * D
            for j in range(D // L):
                rows = plsc.load_gather(tbuf, [base + cols[j]])
                plsc.addupdate(xref.at[t, pl.ds(j * L, L)], rows)

    issue_p(0, p0, sp0)
    issue_p(1, p1, sp1)
    issue_xa(0)
    issue_xb(0)

    def pair_body(p, carry):
        k0 = 2 * p
        k1 = k0 + 1
        wait_p(p0, sp0)
        wait_xa()
        add_half(xa, p0, H1, 0)
        issue_oa(k0)
        wait_xb()
        add_half(xb, p0, H2, H1)
        issue_ob(k0)

        @pl.when(valid(k0 + 2))
        def _():
            issue_p(k0 + 2, p0, sp0)

        wait_oa()
        issue_xa(k1)
        wait_p(p1, sp1)
        wait_xa()
        add_half(xa, p1, H1, 0)
        issue_oa(k1)
        wait_ob()
        issue_xb(k1)
        wait_xb()
        add_half(xb, p1, H2, H1)
        issue_ob(k1)

        @pl.when(valid(k1 + 2))
        def _():
            issue_p(k1 + 2, p1, sp1)

        @pl.when(valid(k0 + 2))
        def _():
            wait_oa()
            issue_xa(k0 + 2)
            wait_ob()
            issue_xb(k0 + 2)
        return carry

    lax.fori_loop(0, KFULL // 2, pair_body, 0)

    @pl.when(wid < EXTRA)
    def _extra():
        wait_p(p0, sp0)
        wait_xa()
        add_half(xa, p0, H1, 0)
        issue_oa(KFULL)
        wait_xb()
        add_half(xb, p0, H2, H1)
        issue_ob(KFULL)

    wait_oa()
    wait_ob()


@jax.jit
def kernel(x, emb_table, pos_matrix):
    table1 = emb_table.reshape(TABLE_ROWS * D)
    mesh = plsc.VectorSubcoreMesh(core_axis_name="c", subcore_axis_name="s")
    run = functools.partial(
        pl.kernel,
        mesh=mesh,
        out_type=jax.ShapeDtypeStruct((ROWS, COLS, D), jnp.float32),
        compiler_params=pltpu.CompilerParams(
            needs_layout_passes=False, use_tc_tiling_on_sc=True),
        scratch_types=[
            pltpu.VMEM((TABLE_ROWS * D,), jnp.float32),
            pltpu.VMEM((H1, D), jnp.float32),
            pltpu.VMEM((H2, D), jnp.float32),
            pltpu.VMEM((COLS,), jnp.int32),
            pltpu.VMEM((COLS,), jnp.int32),
        ] + [pltpu.SemaphoreType.DMA] * 7,
    )(_body)
    return run(x, table1, pos_matrix)
